# baseline (speedup 1.0000x reference)
import functools

import jax
import jax.numpy as jnp
from jax import lax
from jax.experimental import pallas as pl
from jax.experimental.pallas import tpu as pltpu
from jax.experimental.pallas import tpu_sc as plsc

_VOCAB = 1000000
_D = 64
_SCALE = float(_D) ** 0.5

_NC = 2
_NS = 16
_NW = _NC * _NS

_BATCH = 16384
_SEQ = 50
_B = _BATCH * _SEQ
_C = 128
_NCHUNK_TOT = _B // _C
_PER_W = _NCHUNK_TOT // _NW
_NB = 4
_IT = _BATCH // _C
_ROWS_PER_J = _D // 8 * _IT * 8


def _sc_body(table_hbm, idx_hbm, out_hbm, idx_v, rows_v, tb_v, *sems):
    gsems = sems[:_NB]
    osems = sems[_NB:]

    c = lax.axis_index("c")
    s = lax.axis_index("s")
    wid = s * _NC + c
    d0 = wid * _PER_W

    pltpu.sync_copy(idx_hbm.at[pl.ds(d0, _PER_W)], idx_v)

    lane = lax.iota(jnp.int32, 16)

    def issue_gather(m, b):
        pltpu.async_copy(table_hbm.at[idx_v.at[m]], rows_v.at[b], gsems[b])

    def drain_gather(b):
        pltpu.make_async_copy(
            table_hbm.at[pl.ds(0, _C)], rows_v.at[b], gsems[b]
        ).wait()

    def transcale(b):
        @plsc.parallel_loop(0, _D, 1, unroll=2)
        def _(cc):
            col = jnp.zeros((16,), jnp.int32) + cc
            for h in range(_C // 16):
                vals = plsc.load_gather(rows_v.at[b], [lane + h * 16, col])
                tb_v[b, cc, pl.ds(h * 16, 16)] = vals * _SCALE

    def issue_out(m, b):
        d = d0 + m
        base = (
            lax.shift_right_logical(d, 7) * _ROWS_PER_J
            + lax.bitwise_and(d, _IT - 1) * 8
        )
        for tr in range(_D // 8):
            pltpu.async_copy(
                tb_v.at[b].at[pl.ds(tr * 8, 8)],
                out_hbm.at[pl.ds(base + tr * (_IT * 8), 8)],
                osems[b],
            )

    def drain_out(b):
        pltpu.make_async_copy(
            tb_v.at[b], out_hbm.at[pl.ds(0, _D)], osems[b]
        ).wait()

    for b in range(_NB):
        issue_gather(jnp.int32(b), b)

    for b in range(_NB):
        m = jnp.int32(b)
        drain_gather(b)
        transcale(b)
        issue_out(m, b)
        issue_gather(m + _NB, b)

    def outer(o, carry):
        for b in range(_NB):
            m = o * _NB + b
            drain_out(b)
            drain_gather(b)
            transcale(b)
            issue_out(m, b)
            issue_gather(m + _NB, b)
        return carry

    lax.fori_loop(1, _PER_W // _NB - 1, outer, 0)

    for b in range(_NB):
        m = jnp.int32(_PER_W - _NB + b)
        drain_out(b)
        drain_gather(b)
        transcale(b)
        issue_out(m, b)
    for b in range(_NB):
        drain_out(b)


@jax.jit
def _embed(table, idx2d):
    mesh = plsc.VectorSubcoreMesh(core_axis_name="c", subcore_axis_name="s")
    k = functools.partial(
        pl.kernel,
        out_type=jax.ShapeDtypeStruct((_SEQ * _ROWS_PER_J, _C), jnp.float32),
        mesh=mesh,
        scratch_types=[
            pltpu.VMEM((_PER_W, _C), jnp.int32),
            pltpu.VMEM((_NB, _C, _D), jnp.float32),
            pltpu.VMEM((_NB, _D, _C), jnp.float32),
        ]
        + [pltpu.SemaphoreType.DMA] * (2 * _NB),
        compiler_params=pltpu.CompilerParams(
            use_tc_tiling_on_sc=False, needs_layout_passes=False
        ),
    )(_sc_body)
    return k(table, idx2d)


def kernel(xb, table):
    idx2d = jnp.transpose(xb).astype(jnp.int32).reshape(_NCHUNK_TOT, _C)
    flat = _embed(table, idx2d)
    a = flat.reshape(_SEQ, _D // 8, _IT, 8, _C)
    return a.transpose(2, 4, 0, 1, 3).reshape(_BATCH, _SEQ, _D)
